# Initial kernel scaffold; baseline (speedup 1.0000x reference)
import jax
import jax.numpy as jnp
from jax.experimental import pallas as pl


def kernel(x, emb):
    raise NotImplementedError("write your pallas kernel here")


# speedup vs baseline: 2.6811x; 2.6811x over previous
import functools

import jax
import jax.numpy as jnp
from jax import lax
from jax.experimental import pallas as pl
from jax.experimental.pallas import tpu as pltpu
from jax.experimental.pallas import tpu_sc as plsc

NC = 2
NS = 16
NW = NC * NS

BATCH = 128
GROUP = 8


@functools.lru_cache(maxsize=None)
def _make_gather(n_rows: int, dim: int):
    assert n_rows % (NW * GROUP) == 0
    rows_per_w = n_rows // NW
    n_groups = rows_per_w // GROUP

    mesh = plsc.VectorSubcoreMesh(core_axis_name="c", subcore_axis_name="s")

    @functools.partial(
        pl.kernel,
        mesh=mesh,
        out_type=jax.ShapeDtypeStruct((n_rows, BATCH, dim), jnp.float32),
        scratch_types=[
            pltpu.VMEM((GROUP, BATCH), jnp.int32),
            pltpu.VMEM((GROUP, BATCH, dim), jnp.float32),
            pltpu.SemaphoreType.DMA,
        ],
        compiler_params=pltpu.CompilerParams(use_tc_tiling_on_sc=False),
    )
    def k(emb_hbm, xg_hbm, out_hbm, idx_v, rows_v, sem):
        wid = lax.axis_index("s") * NC + lax.axis_index("c")
        row_base = wid * rows_per_w

        def group_body(g, carry):
            gbase = row_base + g * GROUP
            pltpu.sync_copy(xg_hbm.at[pl.ds(gbase, GROUP)], idx_v)
            handles = [
                pltpu.async_copy(emb_hbm.at[idx_v.at[j]], rows_v.at[j], sem)
                for j in range(GROUP)
            ]
            for h in handles:
                h.wait()
            pltpu.sync_copy(rows_v, out_hbm.at[pl.ds(gbase, GROUP)])
            return carry

        lax.fori_loop(0, n_groups, group_body, 0)

    return k


def kernel(x, emb):
    b, s = x.shape
    v, d = emb.shape
    n = b * s
    assert n % BATCH == 0
    d_pad = (d + 15) // 16 * 16
    emb_p = jnp.pad(emb, ((0, 0), (0, d_pad - d))) if d_pad != d else emb
    xg = x.reshape(n // BATCH, BATCH).astype(jnp.int32)
    out = _make_gather(n // BATCH, d_pad)(emb_p.astype(jnp.float32), xg)
    return out[:, :, :d].reshape(b, s, d)
